# Initial kernel scaffold; baseline (speedup 1.0000x reference)
import functools

import jax
import jax.numpy as jnp
from jax import lax
from jax.experimental import pallas as pl
from jax.experimental.pallas import tpu as pltpu
from jax.experimental.pallas import tpu_sc as plsc

_B = 16384
_L = 20
_E = 64
_N = _B * _L
_CHUNK = 128
_NW = 32
_CPW = _N // (_CHUNK * _NW)


def _hash_embed_body(ids_hbm, hcol0_hbm, hcol1_hbm, w_hbm, pcol0_hbm,
                     pcol1_hbm, out_hbm,
                     ids_v, h0_v, h1_v, p0_v, p1_v, pv0_v, pv1_v,
                     w0_v, w1_v, outb_v, sem_a, sem_b, sem_out):
  wid = lax.axis_index("s") * 2 + lax.axis_index("c")

  pltpu.sync_copy(ids_hbm.at[pl.ds(wid * _CPW, _CPW)], ids_v)

  def chunk_body(c, carry):
    idx = ids_v.at[c]

    c0 = pltpu.async_copy(hcol0_hbm.at[idx], h0_v, sem_a)
    c1 = pltpu.async_copy(hcol1_hbm.at[idx], h1_v, sem_a)
    c2 = pltpu.async_copy(pcol0_hbm.at[idx], p0_v, sem_a)
    c3 = pltpu.async_copy(pcol1_hbm.at[idx], p1_v, sem_a)
    c0.wait(); c1.wait(); c2.wait(); c3.wait()

    d0 = pltpu.async_copy(w_hbm.at[h0_v], w0_v, sem_b)
    d1 = pltpu.async_copy(w_hbm.at[h1_v], w1_v, sem_b)
    d2 = pltpu.async_copy(pcol0_hbm.at[h0_v], pv0_v, sem_b)
    d3 = pltpu.async_copy(pcol1_hbm.at[h1_v], pv1_v, sem_b)
    d0.wait(); d1.wait(); d2.wait(); d3.wait()

    def row_body(r, carry2):
      p0s = p0_v[r]
      p1s = p1_v[r]
      for j in range(_E // 16):
        sl = pl.ds(j * 16, 16)
        outb_v[r, sl] = w0_v[r, sl] * p0s + w1_v[r, sl] * p1s
      return carry2

    lax.fori_loop(0, _CHUNK, row_body, 0)

    cols64 = jnp.full((16,), _E, jnp.int32)
    cols65 = jnp.full((16,), _E + 1, jnp.int32)
    for g in range(_CHUNK // 16):
      rows = lax.iota(jnp.int32, 16) + g * 16
      plsc.store_scatter(outb_v, [rows, cols64], pv0_v[pl.ds(g * 16, 16)])
      plsc.store_scatter(outb_v, [rows, cols65], pv1_v[pl.ds(g * 16, 16)])

    row0 = (wid * _CPW + c) * _CHUNK
    pltpu.async_copy(outb_v, out_hbm.at[pl.ds(row0, _CHUNK)], sem_out).wait()
    return carry

  lax.fori_loop(0, _CPW, chunk_body, 0)


def kernel(words_as_ids, hash_table, W, P):
  ids = words_as_ids.reshape(_N // _CHUNK, _CHUNK).astype(jnp.int32)
  hcol0 = hash_table[:, 0].astype(jnp.int32)
  hcol1 = hash_table[:, 1].astype(jnp.int32)
  pcol0 = P[:, 0]
  pcol1 = P[:, 1]

  mesh = plsc.VectorSubcoreMesh(core_axis_name="c", subcore_axis_name="s")
  run = pl.kernel(
      _hash_embed_body,
      out_type=jax.ShapeDtypeStruct((_N, _E + 2), jnp.float32),
      mesh=mesh,
      scratch_types=[
          pltpu.VMEM((_CPW, _CHUNK), jnp.int32),
          pltpu.VMEM((_CHUNK,), jnp.int32),
          pltpu.VMEM((_CHUNK,), jnp.int32),
          pltpu.VMEM((_CHUNK,), jnp.float32),
          pltpu.VMEM((_CHUNK,), jnp.float32),
          pltpu.VMEM((_CHUNK,), jnp.float32),
          pltpu.VMEM((_CHUNK,), jnp.float32),
          pltpu.VMEM((_CHUNK, _E), jnp.float32),
          pltpu.VMEM((_CHUNK, _E), jnp.float32),
          pltpu.VMEM((_CHUNK, _E + 2), jnp.float32),
          pltpu.SemaphoreType.DMA,
          pltpu.SemaphoreType.DMA,
          pltpu.SemaphoreType.DMA,
      ],
  )
  out = run(ids, hcol0, hcol1, W, pcol0, pcol1)
  return out.reshape(_B, _L, _E + 2)


# speedup vs baseline: 3.1528x; 3.1528x over previous
import functools

import jax
import jax.numpy as jnp
from jax import lax
from jax.experimental import pallas as pl
from jax.experimental.pallas import tpu as pltpu
from jax.experimental.pallas import tpu_sc as plsc

_B = 16384
_L = 20
_E = 64
_N = _B * _L
_CHUNK = 128
_NW = 32
_CPW = _N // (_CHUNK * _NW)


def _hash_embed_body(ids_hbm, hcol0_hbm, hcol1_hbm, w_hbm, pcol0_hbm,
                     pcol1_hbm, out_hbm,
                     ids_v, h0_v, h1_v, h0h_v, h1h_v, p0_v, p1_v,
                     pv0_v, pv1_v, w0_v, w1_v, outb_v,
                     sem_a, sem_b, sem_out):
  wid = lax.axis_index("s") * 2 + lax.axis_index("c")

  pltpu.sync_copy(ids_hbm.at[pl.ds(wid * _CPW, _CPW)], ids_v)

  def chunk_body(c, carry):
    idx = ids_v.at[c]

    c0 = pltpu.async_copy(hcol0_hbm.at[idx], h0_v, sem_a)
    c1 = pltpu.async_copy(hcol1_hbm.at[idx], h1_v, sem_a)
    c2 = pltpu.async_copy(pcol0_hbm.at[idx], p0_v, sem_a)
    c3 = pltpu.async_copy(pcol1_hbm.at[idx], p1_v, sem_a)
    c0.wait(); c1.wait(); c2.wait(); c3.wait()

    for g in range(_CHUNK // 16):
      sl = pl.ds(g * 16, 16)
      h0h_v[sl] = lax.shift_right_logical(h0_v[sl], 1)
      h1h_v[sl] = lax.shift_right_logical(h1_v[sl], 1)

    d0 = pltpu.async_copy(w_hbm.at[h0h_v], w0_v, sem_b)
    d1 = pltpu.async_copy(w_hbm.at[h1h_v], w1_v, sem_b)
    d2 = pltpu.async_copy(pcol0_hbm.at[h0_v], pv0_v, sem_b)
    d3 = pltpu.async_copy(pcol1_hbm.at[h1_v], pv1_v, sem_b)
    d0.wait(); d1.wait(); d2.wait(); d3.wait()

    lane0 = lax.iota(jnp.int32, 16) == 0

    def group_body(g, carry2):
      base = g * 16
      p0vec = p0_v[pl.ds(base, 16)]
      p1vec = p1_v[pl.ds(base, 16)]
      pv0vec = pv0_v[pl.ds(base, 16)]
      pv1vec = pv1_v[pl.ds(base, 16)]
      off0vec = (h0_v[pl.ds(base, 16)] & 1) * _E
      off1vec = (h1_v[pl.ds(base, 16)] & 1) * _E
      for r16 in range(16):
        r = base + r16
        p0s = p0vec[r16]
        p1s = p1vec[r16]
        off0 = off0vec[r16]
        off1 = off1vec[r16]
        r66 = r * (_E + 2)
        for j in range(_E // 16):
          outb_v[pl.ds(r66 + j * 16, 16)] = (
              w0_v[r, pl.ds(off0 + j * 16, 16)] * p0s
              + w1_v[r, pl.ds(off1 + j * 16, 16)] * p1s)
        pvpair = jnp.where(lane0, pv0vec[r16], pv1vec[r16])
        outb_v[pl.ds(r66 + _E, 16)] = pvpair
      return carry2

    lax.fori_loop(0, _CHUNK // 16, group_body, 0)

    elem0 = (wid * _CPW + c) * _CHUNK * (_E + 2)
    pltpu.async_copy(
        outb_v.at[pl.ds(0, _CHUNK * (_E + 2))],
        out_hbm.at[pl.ds(elem0, _CHUNK * (_E + 2))], sem_out).wait()
    return carry

  lax.fori_loop(0, _CPW, chunk_body, 0)


def kernel(words_as_ids, hash_table, W, P):
  ids = words_as_ids.reshape(_N // _CHUNK, _CHUNK).astype(jnp.int32)
  hcol0 = hash_table[:, 0].astype(jnp.int32)
  hcol1 = hash_table[:, 1].astype(jnp.int32)
  pcol0 = P[:, 0]
  pcol1 = P[:, 1]
  w2 = W.reshape(W.shape[0] // 2, 2 * _E)

  mesh = plsc.VectorSubcoreMesh(core_axis_name="c", subcore_axis_name="s")
  run = pl.kernel(
      _hash_embed_body,
      out_type=jax.ShapeDtypeStruct((_N * (_E + 2),), jnp.float32),
      mesh=mesh,
      scratch_types=[
          pltpu.VMEM((_CPW, _CHUNK), jnp.int32),
          pltpu.VMEM((_CHUNK,), jnp.int32),
          pltpu.VMEM((_CHUNK,), jnp.int32),
          pltpu.VMEM((_CHUNK,), jnp.int32),
          pltpu.VMEM((_CHUNK,), jnp.int32),
          pltpu.VMEM((_CHUNK,), jnp.float32),
          pltpu.VMEM((_CHUNK,), jnp.float32),
          pltpu.VMEM((_CHUNK,), jnp.float32),
          pltpu.VMEM((_CHUNK,), jnp.float32),
          pltpu.VMEM((_CHUNK, 2 * _E), jnp.float32),
          pltpu.VMEM((_CHUNK, 2 * _E), jnp.float32),
          pltpu.VMEM((_CHUNK * (_E + 2) + 16,), jnp.float32),
          pltpu.SemaphoreType.DMA,
          pltpu.SemaphoreType.DMA,
          pltpu.SemaphoreType.DMA,
      ],
  )
  out = run(ids, hcol0, hcol1, w2, pcol0, pcol1)
  return out.reshape(_B, _L, _E + 2)


# speedup vs baseline: 4.4839x; 1.4222x over previous
import jax
import jax.numpy as jnp
from jax import lax
from jax.experimental import pallas as pl
from jax.experimental.pallas import tpu as pltpu
from jax.experimental.pallas import tpu_sc as plsc

_B = 16384
_L = 20
_E = 64
_N = _B * _L
_CHUNK = 128
_NW = 32
_CPW = _N // (_CHUNK * _NW)
_OSLAB = _CHUNK * (_E + 2)


def _hash_embed_body(ids_hbm, hcol0_hbm, hcol1_hbm, w_hbm, pcol0_hbm,
                     pcol1_hbm, out_hbm,
                     ids_v, h0_v, h1_v, h0h_v, h1h_v, p0_v, p1_v,
                     pv0_v, pv1_v, w0_v, w1_v, outb_v,
                     sem_a, sem_b, sem_out):
  wid = lax.axis_index("s") * 2 + lax.axis_index("c")

  pltpu.sync_copy(ids_hbm.at[pl.ds(wid * _CPW, _CPW)], ids_v)

  lane0 = lax.iota(jnp.int32, 16) == 0

  def start_a(c):
    s = lax.rem(c, 3)
    idx = ids_v.at[c]
    pltpu.async_copy(hcol0_hbm.at[idx], h0_v.at[s], sem_a)
    pltpu.async_copy(hcol1_hbm.at[idx], h1_v.at[s], sem_a)
    pltpu.async_copy(pcol0_hbm.at[idx], p0_v.at[s], sem_a)
    pltpu.async_copy(pcol1_hbm.at[idx], p1_v.at[s], sem_a)

  def wait_a():
    idx = ids_v.at[0]
    pltpu.make_async_copy(hcol0_hbm.at[idx], h0_v.at[0], sem_a).wait()
    pltpu.make_async_copy(hcol1_hbm.at[idx], h1_v.at[0], sem_a).wait()
    pltpu.make_async_copy(pcol0_hbm.at[idx], p0_v.at[0], sem_a).wait()
    pltpu.make_async_copy(pcol1_hbm.at[idx], p1_v.at[0], sem_a).wait()

  def start_b(c):
    s = lax.rem(c, 3)
    b = c & 1
    for g in range(_CHUNK // 16):
      sl = pl.ds(g * 16, 16)
      h0h_v[s, sl] = lax.shift_right_logical(h0_v[s, sl], 1)
      h1h_v[s, sl] = lax.shift_right_logical(h1_v[s, sl], 1)
    pltpu.async_copy(w_hbm.at[h0h_v.at[s]], w0_v.at[b], sem_b)
    pltpu.async_copy(w_hbm.at[h1h_v.at[s]], w1_v.at[b], sem_b)
    pltpu.async_copy(pcol0_hbm.at[h0_v.at[s]], pv0_v.at[b], sem_b)
    pltpu.async_copy(pcol1_hbm.at[h1_v.at[s]], pv1_v.at[b], sem_b)

  def wait_b():
    pltpu.make_async_copy(w_hbm.at[h0h_v.at[0]], w0_v.at[0], sem_b).wait()
    pltpu.make_async_copy(w_hbm.at[h1h_v.at[0]], w1_v.at[0], sem_b).wait()
    pltpu.make_async_copy(pcol0_hbm.at[h0_v.at[0]], pv0_v.at[0], sem_b).wait()
    pltpu.make_async_copy(pcol1_hbm.at[h1_v.at[0]], pv1_v.at[0], sem_b).wait()

  def start_d(c):
    boff = (c & 1) * (_OSLAB + 16)
    elem0 = (wid * _CPW + c) * _OSLAB
    pltpu.async_copy(outb_v.at[pl.ds(boff, _OSLAB)],
                     out_hbm.at[pl.ds(elem0, _OSLAB)], sem_out)

  def wait_d():
    pltpu.make_async_copy(outb_v.at[pl.ds(0, _OSLAB)],
                          out_hbm.at[pl.ds(0, _OSLAB)], sem_out).wait()

  def compute(c):
    s = lax.rem(c, 3)
    b = c & 1
    boff = b * (_OSLAB + 16)

    def group_body(g, carry2):
      base = g * 16
      p0vec = p0_v[s, pl.ds(base, 16)]
      p1vec = p1_v[s, pl.ds(base, 16)]
      pv0vec = pv0_v[b, pl.ds(base, 16)]
      pv1vec = pv1_v[b, pl.ds(base, 16)]
      off0vec = (h0_v[s, pl.ds(base, 16)] & 1) * _E
      off1vec = (h1_v[s, pl.ds(base, 16)] & 1) * _E
      for r16 in range(16):
        r = base + r16
        p0s = p0vec[r16]
        p1s = p1vec[r16]
        off0 = off0vec[r16]
        off1 = off1vec[r16]
        r66 = boff + r * (_E + 2)
        for j in range(_E // 16):
          outb_v[pl.ds(r66 + j * 16, 16)] = (
              w0_v[b, r, pl.ds(off0 + j * 16, 16)] * p0s
              + w1_v[b, r, pl.ds(off1 + j * 16, 16)] * p1s)
        pvpair = jnp.where(lane0, pv0vec[r16], pv1vec[r16])
        outb_v[pl.ds(r66 + _E, 16)] = pvpair
      return carry2

    lax.fori_loop(0, _CHUNK // 16, group_body, 0)

  start_a(jnp.int32(0))
  start_a(jnp.int32(1))
  wait_a()
  start_b(jnp.int32(0))

  def chunk_body(c, carry):
    @pl.when(c < _CPW - 2)
    def _():
      start_a(c + 2)

    @pl.when(c < _CPW - 1)
    def _():
      wait_a()
      start_b(c + 1)

    wait_b()

    @pl.when(c >= 2)
    def _():
      wait_d()

    compute(c)
    start_d(c)
    return carry

  lax.fori_loop(0, _CPW, chunk_body, 0)
  wait_d()
  wait_d()


def kernel(words_as_ids, hash_table, W, P):
  ids = words_as_ids.reshape(_N // _CHUNK, _CHUNK).astype(jnp.int32)
  hcol0 = hash_table[:, 0].astype(jnp.int32)
  hcol1 = hash_table[:, 1].astype(jnp.int32)
  pcol0 = P[:, 0]
  pcol1 = P[:, 1]
  w2 = W.reshape(W.shape[0] // 2, 2 * _E)

  mesh = plsc.VectorSubcoreMesh(core_axis_name="c", subcore_axis_name="s")
  run = pl.kernel(
      _hash_embed_body,
      out_type=jax.ShapeDtypeStruct((_N * (_E + 2),), jnp.float32),
      mesh=mesh,
      scratch_types=[
          pltpu.VMEM((_CPW, _CHUNK), jnp.int32),
          pltpu.VMEM((3, _CHUNK), jnp.int32),
          pltpu.VMEM((3, _CHUNK), jnp.int32),
          pltpu.VMEM((3, _CHUNK), jnp.int32),
          pltpu.VMEM((3, _CHUNK), jnp.int32),
          pltpu.VMEM((3, _CHUNK), jnp.float32),
          pltpu.VMEM((3, _CHUNK), jnp.float32),
          pltpu.VMEM((2, _CHUNK), jnp.float32),
          pltpu.VMEM((2, _CHUNK), jnp.float32),
          pltpu.VMEM((2, _CHUNK, 2 * _E), jnp.float32),
          pltpu.VMEM((2, _CHUNK, 2 * _E), jnp.float32),
          pltpu.VMEM((2 * (_OSLAB + 16),), jnp.float32),
          pltpu.SemaphoreType.DMA,
          pltpu.SemaphoreType.DMA,
          pltpu.SemaphoreType.DMA,
      ],
  )
  out = run(ids, hcol0, hcol1, w2, pcol0, pcol1)
  return out.reshape(_B, _L, _E + 2)


# speedup vs baseline: 4.4864x; 1.0006x over previous
import jax
import jax.numpy as jnp
from jax import lax
from jax.experimental import pallas as pl
from jax.experimental.pallas import tpu as pltpu
from jax.experimental.pallas import tpu_sc as plsc

_B = 16384
_L = 20
_E = 64
_N = _B * _L
_CHUNK = 128
_NW = 32
_CPW = _N // (_CHUNK * _NW)
_OSLAB = _CHUNK * (_E + 2)


def _hash_embed_body(ids_hbm, hcol0_hbm, hcol1_hbm, w_hbm, pcol0_hbm,
                     pcol1_hbm, out_hbm,
                     ids_v, h0_v, h1_v, h0h_v, h1h_v, p0_v, p1_v,
                     pv0_v, pv1_v, w0_v, w1_v, outb_v,
                     sem_a, sem_b, sem_out):
  wid = lax.axis_index("s") * 2 + lax.axis_index("c")

  pltpu.sync_copy(ids_hbm.at[pl.ds(wid * _CPW, _CPW)], ids_v)

  lane0 = lax.iota(jnp.int32, 16) == 0

  def start_a(c):
    s = lax.rem(c, 3)
    idx = ids_v.at[c]
    pltpu.async_copy(hcol0_hbm.at[idx], h0_v.at[s], sem_a)
    pltpu.async_copy(hcol1_hbm.at[idx], h1_v.at[s], sem_a)
    pltpu.async_copy(pcol0_hbm.at[idx], p0_v.at[s], sem_a)
    pltpu.async_copy(pcol1_hbm.at[idx], p1_v.at[s], sem_a)

  def wait_a():
    idx = ids_v.at[0]
    pltpu.make_async_copy(hcol0_hbm.at[idx], h0_v.at[0], sem_a).wait()
    pltpu.make_async_copy(hcol1_hbm.at[idx], h1_v.at[0], sem_a).wait()
    pltpu.make_async_copy(pcol0_hbm.at[idx], p0_v.at[0], sem_a).wait()
    pltpu.make_async_copy(pcol1_hbm.at[idx], p1_v.at[0], sem_a).wait()

  def start_b(c):
    s = lax.rem(c, 3)
    b = c & 1
    for g in range(_CHUNK // 16):
      sl = pl.ds(g * 16, 16)
      h0h_v[s, sl] = lax.shift_right_logical(h0_v[s, sl], 1)
      h1h_v[s, sl] = lax.shift_right_logical(h1_v[s, sl], 1)
    pltpu.async_copy(w_hbm.at[h0h_v.at[s]], w0_v.at[b], sem_b)
    pltpu.async_copy(w_hbm.at[h1h_v.at[s]], w1_v.at[b], sem_b)
    pltpu.async_copy(pcol0_hbm.at[h0_v.at[s]], pv0_v.at[b], sem_b)
    pltpu.async_copy(pcol1_hbm.at[h1_v.at[s]], pv1_v.at[b], sem_b)

  def wait_b():
    pltpu.make_async_copy(w_hbm.at[h0h_v.at[0]], w0_v.at[0], sem_b).wait()
    pltpu.make_async_copy(w_hbm.at[h1h_v.at[0]], w1_v.at[0], sem_b).wait()
    pltpu.make_async_copy(pcol0_hbm.at[h0_v.at[0]], pv0_v.at[0], sem_b).wait()
    pltpu.make_async_copy(pcol1_hbm.at[h1_v.at[0]], pv1_v.at[0], sem_b).wait()

  def start_d(c):
    boff = (c & 1) * (_OSLAB + 16)
    elem0 = (wid * _CPW + c) * _OSLAB
    pltpu.async_copy(outb_v.at[pl.ds(boff, _OSLAB)],
                     out_hbm.at[pl.ds(elem0, _OSLAB)], sem_out)

  def wait_d():
    pltpu.make_async_copy(outb_v.at[pl.ds(0, _OSLAB)],
                          out_hbm.at[pl.ds(0, _OSLAB)], sem_out).wait()

  def compute(c):
    s = lax.rem(c, 3)
    b = c & 1
    boff = b * (_OSLAB + 16)

    def group_body(g, carry2):
      base = g * 16
      p0vec = p0_v[s, pl.ds(base, 16)]
      p1vec = p1_v[s, pl.ds(base, 16)]
      pv0vec = pv0_v[b, pl.ds(base, 16)]
      pv1vec = pv1_v[b, pl.ds(base, 16)]
      off0vec = (h0_v[s, pl.ds(base, 16)] & 1) * _E
      off1vec = (h1_v[s, pl.ds(base, 16)] & 1) * _E
      for r16 in range(16):
        r = base + r16
        p0s = p0vec[r16]
        p1s = p1vec[r16]
        off0 = off0vec[r16]
        off1 = off1vec[r16]
        r66 = boff + r * (_E + 2)
        for j in range(_E // 16):
          outb_v[pl.ds(r66 + j * 16, 16)] = (
              w0_v[b, r, pl.ds(off0 + j * 16, 16)] * p0s
              + w1_v[b, r, pl.ds(off1 + j * 16, 16)] * p1s)
        pvpair = jnp.where(lane0, pv0vec[r16], pv1vec[r16])
        outb_v[pl.ds(r66 + _E, 16)] = pvpair
      return carry2

    lax.fori_loop(0, _CHUNK // 16, group_body, 0)

  start_a(jnp.int32(0))
  start_a(jnp.int32(1))
  wait_a()
  start_b(jnp.int32(0))

  def chunk_body(c, carry):
    @pl.when(c < _CPW - 2)
    def _():
      start_a(c + 2)

    @pl.when(c < _CPW - 1)
    def _():
      wait_a()
      start_b(c + 1)

    wait_b()

    @pl.when(c >= 2)
    def _():
      wait_d()

    compute(c)
    start_d(c)
    return carry

  lax.fori_loop(0, _CPW, chunk_body, 0)
  wait_d()
  wait_d()


def kernel(words_as_ids, hash_table, W, P):
  ids = words_as_ids.reshape(_N // _CHUNK, _CHUNK).astype(jnp.int32)
  hcol0 = hash_table[:, 0].astype(jnp.int32)
  hcol1 = hash_table[:, 1].astype(jnp.int32)
  pcol0 = P[:, 0]
  pcol1 = P[:, 1]
  w2 = W.reshape(W.shape[0] // 2, 2 * _E)

  mesh = plsc.VectorSubcoreMesh(core_axis_name="c", subcore_axis_name="s")
  run = pl.kernel(
      _hash_embed_body,
      out_type=jax.ShapeDtypeStruct((_N * (_E + 2),), jnp.float32),
      mesh=mesh,
      compiler_params=pltpu.CompilerParams(use_tc_tiling_on_sc=True),
      scratch_types=[
          pltpu.VMEM((_CPW, _CHUNK), jnp.int32),
          pltpu.VMEM((3, _CHUNK), jnp.int32),
          pltpu.VMEM((3, _CHUNK), jnp.int32),
          pltpu.VMEM((3, _CHUNK), jnp.int32),
          pltpu.VMEM((3, _CHUNK), jnp.int32),
          pltpu.VMEM((3, _CHUNK), jnp.float32),
          pltpu.VMEM((3, _CHUNK), jnp.float32),
          pltpu.VMEM((2, _CHUNK), jnp.float32),
          pltpu.VMEM((2, _CHUNK), jnp.float32),
          pltpu.VMEM((2, _CHUNK, 2 * _E), jnp.float32),
          pltpu.VMEM((2, _CHUNK, 2 * _E), jnp.float32),
          pltpu.VMEM((2 * (_OSLAB + 16),), jnp.float32),
          pltpu.SemaphoreType.DMA,
          pltpu.SemaphoreType.DMA,
          pltpu.SemaphoreType.DMA,
      ],
  )
  out = run(ids, hcol0, hcol1, w2, pcol0, pcol1)
  return out.reshape(_B, _L, _E + 2)
